# Initial kernel scaffold; baseline (speedup 1.0000x reference)
import jax
import jax.numpy as jnp
from jax.experimental import pallas as pl


def kernel(triangles, points):
    raise NotImplementedError("write your pallas kernel here")


# speedup vs baseline: 180.0676x; 180.0676x over previous
import functools

import jax
import jax.numpy as jnp
from jax.experimental import pallas as pl
from jax.experimental.pallas import tpu as pltpu

_BQ = 128
_BF = 256


def _sdiv(x, y):
    return x / jnp.where(jnp.abs(y) < 1e-12, 1e-12, y)


def _scan_kernel(nf_real, px_r, py_r, pz_r, tri_r, d2_r, cp_r, idx_r):
    fi = pl.program_id(1)
    bf = tri_r.shape[1]

    @pl.when(fi == 0)
    def _init():
        d2_r[...] = jnp.full_like(d2_r, jnp.inf)
        idx_r[...] = jnp.zeros_like(idx_r)
        cp_r[...] = jnp.zeros_like(cp_r)

    ax = tri_r[0:1, :]
    ay = tri_r[1:2, :]
    az = tri_r[2:3, :]
    bx = tri_r[3:4, :]
    by = tri_r[4:5, :]
    bz = tri_r[5:6, :]
    cx = tri_r[6:7, :]
    cy = tri_r[7:8, :]
    cz = tri_r[8:9, :]
    abx = bx - ax
    aby = by - ay
    abz = bz - az
    acx = cx - ax
    acy = cy - ay
    acz = cz - az

    px = px_r[...]
    py = py_r[...]
    pz = pz_r[...]

    apx = px - ax
    apy = py - ay
    apz = pz - az
    d1 = abx * apx + aby * apy + abz * apz
    d2 = acx * apx + acy * apy + acz * apz
    bpx = px - bx
    bpy = py - by
    bpz = pz - bz
    d3 = abx * bpx + aby * bpy + abz * bpz
    d4 = acx * bpx + acy * bpy + acz * bpz
    cpx = px - cx
    cpy = py - cy
    cpz = pz - cz
    d5 = abx * cpx + aby * cpy + abz * cpz
    d6 = acx * cpx + acy * cpy + acz * cpz

    va = d3 * d6 - d5 * d4
    vb = d5 * d2 - d1 * d6
    vc = d1 * d4 - d3 * d2

    denom = va + vb + vc
    v_int = _sdiv(vb, denom)
    w_int = _sdiv(vc, denom)
    rx = ax + abx * v_int + acx * w_int
    ry = ay + aby * v_int + acy * w_int
    rz = az + abz * v_int + acz * w_int

    d43 = d4 - d3
    d56 = d5 - d6
    w_bc = _sdiv(d43, d43 + d56)
    cond_bc = (va <= 0) & (d43 >= 0) & (d56 >= 0)
    rx = jnp.where(cond_bc, bx + (cx - bx) * w_bc, rx)
    ry = jnp.where(cond_bc, by + (cy - by) * w_bc, ry)
    rz = jnp.where(cond_bc, bz + (cz - bz) * w_bc, rz)
    w_ac = _sdiv(d2, d2 - d6)
    cond_ac = (vb <= 0) & (d2 >= 0) & (d6 <= 0)
    rx = jnp.where(cond_ac, ax + acx * w_ac, rx)
    ry = jnp.where(cond_ac, ay + acy * w_ac, ry)
    rz = jnp.where(cond_ac, az + acz * w_ac, rz)
    v_ab = _sdiv(d1, d1 - d3)
    cond_ab = (vc <= 0) & (d1 >= 0) & (d3 <= 0)
    rx = jnp.where(cond_ab, ax + abx * v_ab, rx)
    ry = jnp.where(cond_ab, ay + aby * v_ab, ry)
    rz = jnp.where(cond_ab, az + abz * v_ab, rz)
    cond_c = (d6 >= 0) & (d5 <= d6)
    rx = jnp.where(cond_c, cx, rx)
    ry = jnp.where(cond_c, cy, ry)
    rz = jnp.where(cond_c, cz, rz)
    cond_b = (d3 >= 0) & (d4 <= d3)
    rx = jnp.where(cond_b, bx, rx)
    ry = jnp.where(cond_b, by, ry)
    rz = jnp.where(cond_b, bz, rz)
    cond_a = (d1 <= 0) & (d2 <= 0)
    rx = jnp.where(cond_a, ax, rx)
    ry = jnp.where(cond_a, ay, ry)
    rz = jnp.where(cond_a, az, rz)

    dist = (rx - px) ** 2 + (ry - py) ** 2 + (rz - pz) ** 2

    lane = jax.lax.broadcasted_iota(jnp.int32, (1, bf), 1)
    valid = (fi * bf + lane) < nf_real
    dist = jnp.where(valid, dist, jnp.inf)

    m = jnp.min(dist, axis=1, keepdims=True)
    is_min = dist == m
    loc = jnp.min(jnp.where(is_min, lane, jnp.int32(2**30)), axis=1, keepdims=True)
    onehot = lane == loc
    zero = jnp.zeros_like(dist)
    cpx_b = jnp.sum(jnp.where(onehot, rx, zero), axis=1, keepdims=True)
    cpy_b = jnp.sum(jnp.where(onehot, ry, zero), axis=1, keepdims=True)
    cpz_b = jnp.sum(jnp.where(onehot, rz, zero), axis=1, keepdims=True)
    gidx = fi * bf + loc

    best = d2_r[...]
    better = m < best
    d2_r[...] = jnp.where(better, m, best)
    idx_r[...] = jnp.where(better, gidx, idx_r[...])
    cp_old = cp_r[...]
    cp_new = jnp.concatenate([cpx_b, cpy_b, cpz_b], axis=1)
    cp_r[...] = jnp.where(better, cp_new, cp_old)


def kernel(triangles, points):
    _, nf, _, _ = triangles.shape
    nq = points.shape[1]
    fp = pl.cdiv(nf, _BF) * _BF
    qp = pl.cdiv(nq, _BQ) * _BQ

    tri9 = triangles[0].reshape(nf, 9).T
    tri9 = jnp.pad(tri9, ((0, 7), (0, fp - nf)))
    pts = jnp.pad(points[0], ((0, qp - nq), (0, 0)))
    px = pts[:, 0:1]
    py = pts[:, 1:2]
    pz = pts[:, 2:3]

    grid = (qp // _BQ, fp // _BF)
    d2, cp, idx = pl.pallas_call(
        functools.partial(_scan_kernel, nf),
        grid=grid,
        in_specs=[
            pl.BlockSpec((_BQ, 1), lambda qi, fi: (qi, 0)),
            pl.BlockSpec((_BQ, 1), lambda qi, fi: (qi, 0)),
            pl.BlockSpec((_BQ, 1), lambda qi, fi: (qi, 0)),
            pl.BlockSpec((16, _BF), lambda qi, fi: (0, fi)),
        ],
        out_specs=[
            pl.BlockSpec((_BQ, 1), lambda qi, fi: (qi, 0)),
            pl.BlockSpec((_BQ, 3), lambda qi, fi: (qi, 0)),
            pl.BlockSpec((_BQ, 1), lambda qi, fi: (qi, 0)),
        ],
        out_shape=[
            jax.ShapeDtypeStruct((qp, 1), jnp.float32),
            jax.ShapeDtypeStruct((qp, 3), jnp.float32),
            jax.ShapeDtypeStruct((qp, 1), jnp.int32),
        ],
        compiler_params=pltpu.CompilerParams(
            dimension_semantics=("parallel", "arbitrary"),
        ),
    )(px, py, pz, tri9)

    distances = d2[:nq, 0][None]
    closest_points = cp[:nq][None]
    closest_faces = idx[:nq, 0][None]
    return distances, closest_points, closest_faces


# speedup vs baseline: 239.7443x; 1.3314x over previous
import functools

import jax
import jax.numpy as jnp
from jax import lax
from jax.experimental import pallas as pl
from jax.experimental.pallas import tpu as pltpu
from jax.experimental.pallas import tpu_sc as plsc

_BQ = 256
_BF = 512
_QP = 7168
_BQ3 = 512


def _sdiv(x, y):
    return x / jnp.where(jnp.abs(y) < 1e-12, 1e-12, y)


def _closest_faithful(px, py, pz, ax, ay, az, bx, by, bz, cx, cy, cz):
    abx = bx - ax
    aby = by - ay
    abz = bz - az
    acx = cx - ax
    acy = cy - ay
    acz = cz - az
    apx = px - ax
    apy = py - ay
    apz = pz - az
    d1 = abx * apx + aby * apy + abz * apz
    d2 = acx * apx + acy * apy + acz * apz
    bpx = px - bx
    bpy = py - by
    bpz = pz - bz
    d3 = abx * bpx + aby * bpy + abz * bpz
    d4 = acx * bpx + acy * bpy + acz * bpz
    cpx = px - cx
    cpy = py - cy
    cpz = pz - cz
    d5 = abx * cpx + aby * cpy + abz * cpz
    d6 = acx * cpx + acy * cpy + acz * cpz
    va = d3 * d6 - d5 * d4
    vb = d5 * d2 - d1 * d6
    vc = d1 * d4 - d3 * d2
    denom = va + vb + vc
    v_int = _sdiv(vb, denom)
    w_int = _sdiv(vc, denom)
    rx = ax + abx * v_int + acx * w_int
    ry = ay + aby * v_int + acy * w_int
    rz = az + abz * v_int + acz * w_int
    d43 = d4 - d3
    d56 = d5 - d6
    w_bc = _sdiv(d43, d43 + d56)
    cond = (va <= 0) & (d43 >= 0) & (d56 >= 0)
    rx = jnp.where(cond, bx + (cx - bx) * w_bc, rx)
    ry = jnp.where(cond, by + (cy - by) * w_bc, ry)
    rz = jnp.where(cond, bz + (cz - bz) * w_bc, rz)
    w_ac = _sdiv(d2, d2 - d6)
    cond = (vb <= 0) & (d2 >= 0) & (d6 <= 0)
    rx = jnp.where(cond, ax + acx * w_ac, rx)
    ry = jnp.where(cond, ay + acy * w_ac, ry)
    rz = jnp.where(cond, az + acz * w_ac, rz)
    v_ab = _sdiv(d1, d1 - d3)
    cond = (vc <= 0) & (d1 >= 0) & (d3 <= 0)
    rx = jnp.where(cond, ax + abx * v_ab, rx)
    ry = jnp.where(cond, ay + aby * v_ab, ry)
    rz = jnp.where(cond, az + abz * v_ab, rz)
    cond = (d6 >= 0) & (d5 <= d6)
    rx = jnp.where(cond, cx, rx)
    ry = jnp.where(cond, cy, ry)
    rz = jnp.where(cond, cz, rz)
    cond = (d3 >= 0) & (d4 <= d3)
    rx = jnp.where(cond, bx, rx)
    ry = jnp.where(cond, by, ry)
    rz = jnp.where(cond, bz, rz)
    cond = (d1 <= 0) & (d2 <= 0)
    rx = jnp.where(cond, ax, rx)
    ry = jnp.where(cond, ay, ry)
    rz = jnp.where(cond, az, rz)
    dist = (rx - px) ** 2 + (ry - py) ** 2 + (rz - pz) ** 2
    return rx, ry, rz, dist


def _scan_kernel(nf_real, px_r, py_r, pz_r, tri_r, d2_r, idx_r):
    fi = pl.program_id(1)
    bf = tri_r.shape[1]

    @pl.when(fi == 0)
    def _init():
        d2_r[...] = jnp.full_like(d2_r, jnp.inf)
        idx_r[...] = jnp.zeros_like(idx_r)

    ax = tri_r[0:1, :]
    ay = tri_r[1:2, :]
    az = tri_r[2:3, :]
    bx = tri_r[3:4, :]
    by = tri_r[4:5, :]
    bz = tri_r[5:6, :]
    cx = tri_r[6:7, :]
    cy = tri_r[7:8, :]
    cz = tri_r[8:9, :]
    px = px_r[...]
    py = py_r[...]
    pz = pz_r[...]

    _, _, _, dist = _closest_faithful(px, py, pz, ax, ay, az,
                                      bx, by, bz, cx, cy, cz)

    lane = lax.broadcasted_iota(jnp.int32, (1, bf), 1)
    valid = (fi * bf + lane) < nf_real
    dist = jnp.where(valid, dist, jnp.inf)

    m = jnp.min(dist, axis=1, keepdims=True)
    loc = jnp.min(jnp.where(dist == m, lane, jnp.int32(2**30)),
                  axis=1, keepdims=True)
    gidx = fi * bf + loc

    best = d2_r[...]
    better = m < best
    d2_r[...] = jnp.where(better, m, best)
    idx_r[...] = jnp.where(better, gidx, idx_r[...])


def _stage1(px, py, pz, tri9, nf, fp):
    grid = (_QP // _BQ, fp // _BF)
    return pl.pallas_call(
        functools.partial(_scan_kernel, nf),
        grid=grid,
        in_specs=[
            pl.BlockSpec((_BQ, 1), lambda qi, fi: (qi, 0)),
            pl.BlockSpec((_BQ, 1), lambda qi, fi: (qi, 0)),
            pl.BlockSpec((_BQ, 1), lambda qi, fi: (qi, 0)),
            pl.BlockSpec((16, _BF), lambda qi, fi: (0, fi)),
        ],
        out_specs=[
            pl.BlockSpec((_BQ, 1), lambda qi, fi: (qi, 0)),
            pl.BlockSpec((_BQ, 1), lambda qi, fi: (qi, 0)),
        ],
        out_shape=[
            jax.ShapeDtypeStruct((_QP, 1), jnp.float32),
            jax.ShapeDtypeStruct((_QP, 1), jnp.int32),
        ],
        compiler_params=pltpu.CompilerParams(
            dimension_semantics=("parallel", "arbitrary"),
        ),
    )(px, py, pz, tri9)


def _sc_gather(tri16, irows):
    mesh = plsc.VectorSubcoreMesh(core_axis_name="c", subcore_axis_name="s")

    @functools.partial(
        pl.kernel,
        mesh=mesh,
        out_type=jax.ShapeDtypeStruct((_QP, 128), jnp.float32),
        scratch_types=[
            pltpu.VMEM((112,), jnp.int32),
            pltpu.VMEM((112, 128), jnp.float32),
            pltpu.SemaphoreType.DMA,
        ],
    )
    def k(tri_hbm, idx_hbm, rows_hbm, idx_v, rows_v, sem):
        wid = lax.axis_index("s") * 2 + lax.axis_index("c")
        for j in range(2):
            r = wid * 2 + j
            pltpu.sync_copy(idx_hbm.at[r], idx_v)
            pltpu.async_copy(tri_hbm.at[idx_v], rows_v, sem).wait()
            pltpu.sync_copy(rows_v, rows_hbm.at[pl.ds(r * 112, 112)])

    return k(tri16, irows)


def _final_kernel(px_r, py_r, pz_r, rows_r, cp_r):
    px = px_r[...]
    py = py_r[...]
    pz = pz_r[...]
    rows = rows_r[...]
    rx, ry, rz, _ = _closest_faithful(
        px, py, pz,
        rows[:, 0:1], rows[:, 1:2], rows[:, 2:3],
        rows[:, 3:4], rows[:, 4:5], rows[:, 5:6],
        rows[:, 6:7], rows[:, 7:8], rows[:, 8:9])
    cp_r[...] = jnp.concatenate([rx, ry, rz], axis=1)


def _stage3(px, py, pz, rows):
    grid = (_QP // _BQ3,)
    return pl.pallas_call(
        _final_kernel,
        grid=grid,
        in_specs=[
            pl.BlockSpec((_BQ3, 1), lambda qi: (qi, 0)),
            pl.BlockSpec((_BQ3, 1), lambda qi: (qi, 0)),
            pl.BlockSpec((_BQ3, 1), lambda qi: (qi, 0)),
            pl.BlockSpec((_BQ3, 128), lambda qi: (qi, 0)),
        ],
        out_specs=pl.BlockSpec((_BQ3, 3), lambda qi: (qi, 0)),
        out_shape=jax.ShapeDtypeStruct((_QP, 3), jnp.float32),
    )(px, py, pz, rows)


def kernel(triangles, points):
    _, nf, _, _ = triangles.shape
    nq = points.shape[1]
    fp = pl.cdiv(nf, _BF) * _BF

    tri_flat = triangles[0].reshape(nf, 9)
    tri9 = jnp.pad(tri_flat.T, ((0, 7), (0, fp - nf)))
    tri16 = jnp.pad(tri_flat, ((0, 0), (0, 119)))
    pts = jnp.pad(points[0], ((0, _QP - nq), (0, 0)))
    px = pts[:, 0:1]
    py = pts[:, 1:2]
    pz = pts[:, 2:3]

    d2, idx = _stage1(px, py, pz, tri9, nf, fp)

    rows = _sc_gather(tri16, idx.reshape(64, 112))
    cp = _stage3(px, py, pz, rows)

    distances = d2[:nq, 0][None]
    closest_points = cp[:nq][None]
    closest_faces = idx[:nq, 0][None]
    return distances, closest_points, closest_faces
